# baseline (speedup 1.0000x reference)
import jax
import jax.numpy as jnp
import numpy as np
from jax.experimental import pallas as pl
from jax.experimental.pallas import tpu as pltpu

TOP_K = 8
NUM_EXPERTS = 64
CHUNK = 2048
SLOTS = 4
_MIN32 = np.int32(-2147483648)


def _select_top8(logits_t):
    n = logits_t.shape[1]
    iota = jax.lax.broadcasted_iota(jnp.int32, (NUM_EXPERTS, n), 0)

    bits = jax.lax.bitcast_convert_type(logits_t, jnp.int32)
    key = jnp.where(bits < 0, bits ^ np.int32(0x7FFFFFFF), bits)
    key = (key & np.int32(~63)) | (np.int32(NUM_EXPERTS - 1) - iota)

    top_keys = []
    for _ in range(TOP_K):
        m = jnp.max(key, axis=0, keepdims=True)
        top_keys.append(m)
        key = jnp.where(key == m, _MIN32, key)

    tk = jnp.concatenate(top_keys, axis=0)
    ids = np.int32(NUM_EXPERTS - 1) - (tk & np.int32(63))
    kv = tk & np.int32(~63)
    vbits = jnp.where(kv < 0, kv ^ np.int32(0x7FFFFFFF), kv)
    tv = jax.lax.bitcast_convert_type(vbits, jnp.float32)

    e = jnp.exp(tv - tv[0:1, :])
    return e / jnp.sum(e, axis=0, keepdims=True), ids


def _router_chunk(x_hbm, w_ref, weights_ref, ids_ref, xbuf, sems):
    i = pl.program_id(0)
    nsteps = pl.num_programs(0)

    def _start(c, slot):
        pltpu.make_async_copy(
            x_hbm.at[pl.ds(c * CHUNK, CHUNK), :],
            xbuf.at[slot],
            sems.at[slot],
        ).start()

    @pl.when(i == 0)
    def _():
        for s in range(SLOTS - 1):
            _start(np.int32(s), np.int32(s))

    c_pre = i + (SLOTS - 1)

    @pl.when(c_pre < nsteps)
    def _():
        _start(c_pre, c_pre % SLOTS)

    slot = i % SLOTS
    pltpu.make_async_copy(
        x_hbm.at[pl.ds(i * CHUNK, CHUNK), :],
        xbuf.at[slot],
        sems.at[slot],
    ).wait()

    logits_t = jax.lax.dot_general(
        w_ref[...], xbuf[slot],
        dimension_numbers=(((1,), (1,)), ((), ())),
        preferred_element_type=jnp.float32,
    )

    weights, ids = _select_top8(logits_t)
    weights_ref[...] = weights
    ids_ref[...] = ids


def kernel(hidden_states, gate_w):
    num_tokens, d_model = hidden_states.shape
    grid = (num_tokens // CHUNK,)
    weights_t, ids_t = pl.pallas_call(
        _router_chunk,
        grid=grid,
        in_specs=[
            pl.BlockSpec(memory_space=pl.ANY),
            pl.BlockSpec((NUM_EXPERTS, d_model), lambda i: (0, 0)),
        ],
        out_specs=[
            pl.BlockSpec((TOP_K, CHUNK), lambda i: (0, i)),
            pl.BlockSpec((TOP_K, CHUNK), lambda i: (0, i)),
        ],
        out_shape=[
            jax.ShapeDtypeStruct((TOP_K, num_tokens), jnp.float32),
            jax.ShapeDtypeStruct((TOP_K, num_tokens), jnp.int32),
        ],
        scratch_shapes=[
            pltpu.VMEM((SLOTS, CHUNK, d_model), jnp.float32),
            pltpu.SemaphoreType.DMA((SLOTS,)),
        ],
        compiler_params=pltpu.CompilerParams(
            dimension_semantics=("arbitrary",),
        ),
    )(hidden_states, gate_w)
    return weights_t.T, ids_t.T
